# baseline (speedup 1.0000x reference)
import functools

import jax
import jax.numpy as jnp
from jax import lax
from jax.experimental import pallas as pl
from jax.experimental.pallas import tpu as pltpu
from jax.experimental.pallas import tpu_sc as plsc

N = 10000
D = 128
E = 320000

NC = 2
NS = 16
NW = NC * NS
CB = 128
NBUF = 2
K = NBUF * (-(-E // (NW * CB * NBUF)))
E_PAD = NW * K * CB
L = 16
ROWS = 10112
RPT = ROWS // NS
BN = 1264


def _sc_segment_sum(x, pidx, zeros):
    mesh = plsc.VectorSubcoreMesh(core_axis_name="c", subcore_axis_name="s")

    @functools.partial(
        pl.kernel,
        out_type=jax.ShapeDtypeStruct((NC, ROWS, D), jnp.float32),
        mesh=mesh,
        scratch_types=[
            pltpu.VMEM((K, CB), jnp.int32),
            pltpu.VMEM((K, CB), jnp.int32),
            pltpu.VMEM((CB, D), jnp.float32),
            pltpu.VMEM_SHARED((ROWS, D), jnp.float32),
            pltpu.SemaphoreType.DMA,
        ],
    )
    def seg(x_hbm, pidx_hbm, z_hbm, out_hbm, src_v, dst_v, rows_v,
            acc_sh, sem):
        c = lax.axis_index("c")
        s = lax.axis_index("s")
        w = c * NS + s
        pltpu.sync_copy(z_hbm, acc_sh.at[pl.ds(s * RPT, RPT)])
        pltpu.sync_copy(pidx_hbm.at[w],
                        src_v)

        def unpack(j, carry):
            for q in range(CB // L):
                v = src_v[j, pl.ds(q * L, L)]
                dst_v[j, pl.ds(q * L, L)] = lax.shift_right_logical(v, 16)
                src_v[j, pl.ds(q * L, L)] = v & 0xFFFF
            return carry

        lax.fori_loop(0, K, unpack, 0)
        plsc.subcore_barrier()

        def body(j, carry):
            pltpu.async_copy(x_hbm.at[src_v.at[j]], rows_v, sem).wait()
            pltpu.sync_copy(rows_v, acc_sh.at[dst_v.at[j]], add=True)
            return carry

        lax.fori_loop(0, K, body, 0)
        plsc.subcore_barrier()
        pltpu.sync_copy(acc_sh.at[pl.ds(s * RPT, RPT)],
                        out_hbm.at[c, pl.ds(s * RPT, RPT)])

    return seg(x, pidx, zeros)


def _tc_dense(p, h, W, Wih, Whh, bih, bhh, relu):
    def body(p_ref, h_ref, w_ref, wih_ref, whh_ref, bih_ref, bhh_ref, o_ref):
        agg = jnp.dot(p_ref[0] + p_ref[1], w_ref[...],
                      preferred_element_type=jnp.float32)
        hb = h_ref[...]
        gi = lax.dot_general(agg, wih_ref[...], (((1,), (1,)), ((), ())),
                             preferred_element_type=jnp.float32) + bih_ref[...]
        gh = lax.dot_general(hb, whh_ref[...], (((1,), (1,)), ((), ())),
                             preferred_element_type=jnp.float32) + bhh_ref[...]
        r = jax.nn.sigmoid(gi[:, :D] + gh[:, :D])
        z = jax.nn.sigmoid(gi[:, D:2 * D] + gh[:, D:2 * D])
        n = jnp.tanh(gi[:, 2 * D:] + r * gh[:, 2 * D:])
        out = (1.0 - z) * n + z * hb
        if relu:
            out = jnp.maximum(out, 0.0)
        o_ref[...] = out

    return pl.pallas_call(
        body,
        grid=(ROWS // BN,),
        in_specs=[
            pl.BlockSpec((NC, BN, D), lambda i: (0, i, 0)),
            pl.BlockSpec((BN, D), lambda i: (i, 0)),
            pl.BlockSpec((D, D), lambda i: (0, 0)),
            pl.BlockSpec((3 * D, D), lambda i: (0, 0)),
            pl.BlockSpec((3 * D, D), lambda i: (0, 0)),
            pl.BlockSpec((1, 3 * D), lambda i: (0, 0)),
            pl.BlockSpec((1, 3 * D), lambda i: (0, 0)),
        ],
        out_specs=pl.BlockSpec((BN, D), lambda i: (i, 0)),
        out_shape=jax.ShapeDtypeStruct((ROWS, D), jnp.float32),
    )(p, h, W, Wih, Whh, bih.reshape(1, 3 * D), bhh.reshape(1, 3 * D))


def kernel(x, edge_index, W1, Wih1, Whh1, bih1, bhh1, W2, Wih2, Whh2, bih2,
           bhh2):
    pad = E_PAD - E
    packed = edge_index[0] | (edge_index[1] << 16)
    pidx = jnp.concatenate(
        [packed, jnp.full((pad,), N << 16, jnp.int32)]).reshape(NW, K, CB)
    zeros = jnp.zeros((RPT, D), jnp.float32)
    xp = jnp.concatenate([x, jnp.zeros((ROWS - N, D), jnp.float32)])

    p1 = _sc_segment_sum(xp, pidx, zeros)
    h1 = _tc_dense(p1, xp, W1, Wih1, Whh1, bih1, bhh1, True)
    p2 = _sc_segment_sum(h1, pidx, zeros)
    h2 = _tc_dense(p2, h1, W2, Wih2, Whh2, bih2, bhh2, False)
    return h2[:N].reshape(N * D)


# speedup vs baseline: 1.0605x; 1.0605x over previous
import functools

import jax
import jax.numpy as jnp
from jax import lax
from jax.experimental import pallas as pl
from jax.experimental.pallas import tpu as pltpu
from jax.experimental.pallas import tpu_sc as plsc

N = 10000
D = 128
E = 320000

NC = 2
NS = 16
NW = NC * NS
CB = 128
NBUF = 2
K = NBUF * (-(-E // (NW * CB * NBUF)))
E_PAD = NW * K * CB
L = 16
ROWS = 10112
RPT = ROWS // NS
BN = 1264


def _sc_segment_sum(x, src_p, dst_p, zeros):
    mesh = plsc.VectorSubcoreMesh(core_axis_name="c", subcore_axis_name="s")

    @functools.partial(
        pl.kernel,
        out_type=jax.ShapeDtypeStruct((NC, ROWS, D), jnp.float32),
        mesh=mesh,
        scratch_types=[
            pltpu.VMEM((K, CB), jnp.int32),
            pltpu.VMEM((K, CB), jnp.int32),
            pltpu.VMEM((CB, D), jnp.float32),
            pltpu.VMEM_SHARED((ROWS, D), jnp.float32),
            pltpu.SemaphoreType.DMA,
        ],
    )
    def seg(x_hbm, src_hbm, dst_hbm, z_hbm, out_hbm, src_v, dst_v, rows_v,
            acc_sh, sem):
        c = lax.axis_index("c")
        s = lax.axis_index("s")
        w = c * NS + s
        pltpu.sync_copy(z_hbm, acc_sh.at[pl.ds(s * RPT, RPT)])
        pltpu.sync_copy(src_hbm.at[w], src_v)
        pltpu.sync_copy(dst_hbm.at[w], dst_v)
        plsc.subcore_barrier()

        def body(j, carry):
            pltpu.async_copy(x_hbm.at[src_v.at[j]], rows_v, sem).wait()
            pltpu.sync_copy(rows_v, acc_sh.at[dst_v.at[j]], add=True)
            return carry

        lax.fori_loop(0, K, body, 0)
        plsc.subcore_barrier()
        pltpu.sync_copy(acc_sh.at[pl.ds(s * RPT, RPT)],
                        out_hbm.at[c, pl.ds(s * RPT, RPT)])

    return seg(x, src_p, dst_p, zeros)


def _tc_dense(p, h, W, Wih, Whh, bih, bhh, relu):
    def body(p_ref, h_ref, w_ref, wih_ref, whh_ref, bih_ref, bhh_ref, o_ref):
        agg = jnp.dot(p_ref[0] + p_ref[1], w_ref[...],
                      preferred_element_type=jnp.float32)
        hb = h_ref[...]
        gi = lax.dot_general(agg, wih_ref[...], (((1,), (1,)), ((), ())),
                             preferred_element_type=jnp.float32) + bih_ref[...]
        gh = lax.dot_general(hb, whh_ref[...], (((1,), (1,)), ((), ())),
                             preferred_element_type=jnp.float32) + bhh_ref[...]
        r = jax.nn.sigmoid(gi[:, :D] + gh[:, :D])
        z = jax.nn.sigmoid(gi[:, D:2 * D] + gh[:, D:2 * D])
        n = jnp.tanh(gi[:, 2 * D:] + r * gh[:, 2 * D:])
        out = (1.0 - z) * n + z * hb
        if relu:
            out = jnp.maximum(out, 0.0)
        o_ref[...] = out

    return pl.pallas_call(
        body,
        grid=(ROWS // BN,),
        in_specs=[
            pl.BlockSpec((NC, BN, D), lambda i: (0, i, 0)),
            pl.BlockSpec((BN, D), lambda i: (i, 0)),
            pl.BlockSpec((D, D), lambda i: (0, 0)),
            pl.BlockSpec((3 * D, D), lambda i: (0, 0)),
            pl.BlockSpec((3 * D, D), lambda i: (0, 0)),
            pl.BlockSpec((1, 3 * D), lambda i: (0, 0)),
            pl.BlockSpec((1, 3 * D), lambda i: (0, 0)),
        ],
        out_specs=pl.BlockSpec((BN, D), lambda i: (i, 0)),
        out_shape=jax.ShapeDtypeStruct((ROWS, D), jnp.float32),
    )(p, h, W, Wih, Whh, bih.reshape(1, 3 * D), bhh.reshape(1, 3 * D))


def kernel(x, edge_index, W1, Wih1, Whh1, bih1, bhh1, W2, Wih2, Whh2, bih2,
           bhh2):
    pad = E_PAD - E
    src_p = jnp.concatenate(
        [edge_index[0], jnp.zeros((pad,), jnp.int32)]).reshape(NW, K, CB)
    dst_p = jnp.concatenate(
        [edge_index[1], jnp.full((pad,), N, jnp.int32)]).reshape(NW, K, CB)
    zeros = jnp.zeros((RPT, D), jnp.float32)
    xp = jnp.concatenate([x, jnp.zeros((ROWS - N, D), jnp.float32)])

    p1 = _sc_segment_sum(xp, src_p, dst_p, zeros)
    h1 = _tc_dense(p1, xp, W1, Wih1, Whh1, bih1, bhh1, True)
    p2 = _sc_segment_sum(h1, src_p, dst_p, zeros)
    h2 = _tc_dense(p2, h1, W2, Wih2, Whh2, bih2, bhh2, False)
    return h2[:N].reshape(N * D)


# speedup vs baseline: 1.4737x; 1.3897x over previous
import functools

import jax
import jax.numpy as jnp
from jax import lax
from jax.experimental import pallas as pl
from jax.experimental.pallas import tpu as pltpu
from jax.experimental.pallas import tpu_sc as plsc

N = 10000
D = 128
E = 320000

NC = 2
NS = 16
NW = NC * NS
CB = 128
NBUF = 2
K = -(-E // (NW * CB))
E_PAD = NW * K * CB
L = 16
ROWS = 10240
RPT = ROWS // NS
BN = 1024


def _sc_segment_sum(x, src_p, dst_p, zeros):
    mesh = plsc.VectorSubcoreMesh(core_axis_name="c", subcore_axis_name="s")

    @functools.partial(
        pl.kernel,
        out_type=jax.ShapeDtypeStruct((NC, ROWS, D), jnp.float32),
        mesh=mesh,
        scratch_types=[
            pltpu.VMEM((K, CB), jnp.int32),
            pltpu.VMEM((K, CB), jnp.int32),
            pltpu.VMEM((CB, D), jnp.float32),
            pltpu.VMEM_SHARED((ROWS, D), jnp.float32),
            pltpu.SemaphoreType.DMA,
        ],
    )
    def seg(x_hbm, src_hbm, dst_hbm, z_hbm, out_hbm, src_v, dst_v, rows_v,
            acc_sh, sem):
        c = lax.axis_index("c")
        s = lax.axis_index("s")
        w = c * NS + s
        pltpu.sync_copy(z_hbm, acc_sh.at[pl.ds(s * RPT, RPT)])
        pltpu.sync_copy(src_hbm.at[w], src_v)
        pltpu.sync_copy(dst_hbm.at[w], dst_v)
        plsc.subcore_barrier()

        def body(j, carry):
            pltpu.async_copy(x_hbm.at[src_v.at[j]], rows_v, sem).wait()
            pltpu.sync_copy(rows_v, acc_sh.at[dst_v.at[j]], add=True)
            return carry

        lax.fori_loop(0, K, body, 0)
        plsc.subcore_barrier()
        pltpu.sync_copy(acc_sh.at[pl.ds(s * RPT, RPT)],
                        out_hbm.at[c, pl.ds(s * RPT, RPT)])

    return seg(x, src_p, dst_p, zeros)


def _tc_dense(p, h, W, Wih, Whh, bih, bhh, relu):
    def body(p_ref, h_ref, w_ref, wih_ref, whh_ref, bih_ref, bhh_ref, o_ref):
        agg = jnp.dot(p_ref[0] + p_ref[1], w_ref[...],
                      preferred_element_type=jnp.float32)
        hb = h_ref[...]
        gi = lax.dot_general(agg, wih_ref[...], (((1,), (1,)), ((), ())),
                             preferred_element_type=jnp.float32) + bih_ref[...]
        gh = lax.dot_general(hb, whh_ref[...], (((1,), (1,)), ((), ())),
                             preferred_element_type=jnp.float32) + bhh_ref[...]
        r = jax.nn.sigmoid(gi[:, :D] + gh[:, :D])
        z = jax.nn.sigmoid(gi[:, D:2 * D] + gh[:, D:2 * D])
        n = jnp.tanh(gi[:, 2 * D:] + r * gh[:, 2 * D:])
        out = (1.0 - z) * n + z * hb
        if relu:
            out = jnp.maximum(out, 0.0)
        o_ref[...] = out

    return pl.pallas_call(
        body,
        grid=(ROWS // BN,),
        in_specs=[
            pl.BlockSpec((NC, BN, D), lambda i: (0, i, 0)),
            pl.BlockSpec((BN, D), lambda i: (i, 0)),
            pl.BlockSpec((D, D), lambda i: (0, 0)),
            pl.BlockSpec((3 * D, D), lambda i: (0, 0)),
            pl.BlockSpec((3 * D, D), lambda i: (0, 0)),
            pl.BlockSpec((1, 3 * D), lambda i: (0, 0)),
            pl.BlockSpec((1, 3 * D), lambda i: (0, 0)),
        ],
        out_specs=pl.BlockSpec((BN, D), lambda i: (i, 0)),
        out_shape=jax.ShapeDtypeStruct((ROWS, D), jnp.float32),
    )(p, h, W, Wih, Whh, bih.reshape(1, 3 * D), bhh.reshape(1, 3 * D))


def kernel(x, edge_index, W1, Wih1, Whh1, bih1, bhh1, W2, Wih2, Whh2, bih2,
           bhh2):
    pad = E_PAD - E
    src_p = jnp.concatenate(
        [edge_index[0], jnp.zeros((pad,), jnp.int32)]).reshape(NW, K, CB)
    dst_p = jnp.concatenate(
        [edge_index[1], jnp.full((pad,), N, jnp.int32)]).reshape(NW, K, CB)
    zeros = jnp.zeros((RPT, D), jnp.float32)
    xp = jnp.concatenate([x, jnp.zeros((ROWS - N, D), jnp.float32)])

    p1 = _sc_segment_sum(xp, src_p, dst_p, zeros)
    h1 = _tc_dense(p1, xp, W1, Wih1, Whh1, bih1, bhh1, True)
    p2 = _sc_segment_sum(h1, src_p, dst_p, zeros)
    h2 = _tc_dense(p2, h1, W2, Wih2, Whh2, bih2, bhh2, False)
    return h2[:N].reshape(N * D)


# speedup vs baseline: 1.4822x; 1.0057x over previous
import functools

import jax
import jax.numpy as jnp
from jax import lax
from jax.experimental import pallas as pl
from jax.experimental.pallas import tpu as pltpu
from jax.experimental.pallas import tpu_sc as plsc

N = 10000
D = 128
E = 320000

NC = 2
NS = 16
NW = NC * NS
CB = 128
NBUF = 2
K = -(-E // (NW * CB))
E_PAD = NW * K * CB
L = 16
ROWS = 10240
RPT = ROWS // NS
BN = 1024


def _sc_segment_sum(x, src_p, dst_p, zeros):
    mesh = plsc.VectorSubcoreMesh(core_axis_name="c", subcore_axis_name="s")

    @functools.partial(
        pl.kernel,
        out_type=jax.ShapeDtypeStruct((NC, ROWS, D), jnp.float32),
        mesh=mesh,
        scratch_types=[
            pltpu.VMEM((K, CB), jnp.int32),
            pltpu.VMEM((K, CB), jnp.int32),
            pltpu.VMEM((CB, D), jnp.float32),
            pltpu.VMEM_SHARED((ROWS, D), jnp.float32),
            pltpu.SemaphoreType.DMA,
        ],
    )
    def seg(x_hbm, src_hbm, dst_hbm, z_hbm, out_hbm, src_v, dst_v, rows_v,
            acc_sh, sem):
        c = lax.axis_index("c")
        s = lax.axis_index("s")
        w = c * NS + s
        pltpu.sync_copy(z_hbm, acc_sh.at[pl.ds(s * RPT, RPT)])
        pltpu.sync_copy(src_hbm.at[w], src_v)
        pltpu.sync_copy(dst_hbm.at[w], dst_v)
        plsc.subcore_barrier()

        def body(j, carry):
            pltpu.async_copy(x_hbm.at[src_v.at[j]], rows_v, sem).wait()
            pltpu.sync_copy(rows_v, acc_sh.at[dst_v.at[j]], add=True)
            return carry

        lax.fori_loop(0, K, body, 0)
        plsc.subcore_barrier()
        pltpu.sync_copy(acc_sh.at[pl.ds(s * RPT, RPT)],
                        out_hbm.at[c, pl.ds(s * RPT, RPT)])

    return seg(x, src_p, dst_p, zeros)


def _tc_dense(p, h, W, Wih, Whh, bih, bhh, relu):
    def body(p_ref, h_ref, w_ref, wih_ref, whh_ref, bih_ref, bhh_ref, o_ref):
        agg = jnp.dot(p_ref[0] + p_ref[1], w_ref[...],
                      preferred_element_type=jnp.float32)
        hb = h_ref[...]
        gi = lax.dot_general(agg, wih_ref[...], (((1,), (1,)), ((), ())),
                             preferred_element_type=jnp.float32) + bih_ref[...]
        gh = lax.dot_general(hb, whh_ref[...], (((1,), (1,)), ((), ())),
                             preferred_element_type=jnp.float32) + bhh_ref[...]
        r = jax.nn.sigmoid(gi[:, :D] + gh[:, :D])
        z = jax.nn.sigmoid(gi[:, D:2 * D] + gh[:, D:2 * D])
        n = jnp.tanh(gi[:, 2 * D:] + r * gh[:, 2 * D:])
        out = (1.0 - z) * n + z * hb
        if relu:
            out = jnp.maximum(out, 0.0)
        o_ref[...] = out

    return pl.pallas_call(
        body,
        grid=(ROWS // BN,),
        in_specs=[
            pl.BlockSpec((NC, BN, D), lambda i: (0, i, 0)),
            pl.BlockSpec((BN, D), lambda i: (i, 0)),
            pl.BlockSpec((D, D), lambda i: (0, 0)),
            pl.BlockSpec((3 * D, D), lambda i: (0, 0)),
            pl.BlockSpec((3 * D, D), lambda i: (0, 0)),
            pl.BlockSpec((1, 3 * D), lambda i: (0, 0)),
            pl.BlockSpec((1, 3 * D), lambda i: (0, 0)),
        ],
        out_specs=pl.BlockSpec((BN, D), lambda i: (i, 0)),
        out_shape=jax.ShapeDtypeStruct((ROWS, D), jnp.float32),
    )(p, h, W, Wih, Whh, bih.reshape(1, 3 * D), bhh.reshape(1, 3 * D))


def kernel(x, edge_index, W1, Wih1, Whh1, bih1, bhh1, W2, Wih2, Whh2, bih2,
           bhh2):
    pad = E_PAD - E
    src_p = jnp.concatenate(
        [edge_index[0], jnp.zeros((pad,), jnp.int32)]).reshape(NW, K, CB)
    dst_p = jnp.concatenate(
        [edge_index[1],
         N + (jnp.arange(pad, dtype=jnp.int32) % (ROWS - N))]).reshape(
             NW, K, CB)
    zeros = jnp.zeros((RPT, D), jnp.float32)
    xp = jnp.concatenate([x, jnp.zeros((ROWS - N, D), jnp.float32)])

    p1 = _sc_segment_sum(xp, src_p, dst_p, zeros)
    h1 = _tc_dense(p1, xp, W1, Wih1, Whh1, bih1, bhh1, True)
    p2 = _sc_segment_sum(h1, src_p, dst_p, zeros)
    h2 = _tc_dense(p2, h1, W2, Wih2, Whh2, bih2, bhh2, False)
    return h2[:N].reshape(N * D)
